# baseline (speedup 1.0000x reference)
import functools

import jax
import jax.numpy as jnp
from jax import lax
from jax.experimental import pallas as pl
from jax.experimental.pallas import tpu as pltpu
from jax.experimental.pallas import tpu_sc as plsc

B = 16384
N_FIELDS = 26
VOCAB = 100000
EMB = 32
NUM = 13
H1 = 256
H2 = 128
NUM_PAD = 64

N_ROWS = B * N_FIELDS
NW = 32

VB = 128
NFULL = VOCAB // VB
VTAIL0 = NFULL * VB
KMAX = (NFULL + NW - 1) // NW

G_PER_W = N_ROWS // NW
GCHUNK = 1664
NGCHUNK = G_PER_W // GCHUNK

NCB = 7
J_ROWS = (B // 8) * NCB * 32

_mesh = plsc.VectorSubcoreMesh(core_axis_name="c", subcore_axis_name="s")


@functools.partial(
    pl.kernel,
    mesh=_mesh,
    compiler_params=pltpu.CompilerParams(needs_layout_passes=False),
    out_type=jax.ShapeDtypeStruct((N_FIELDS * VOCAB * EMB,), jnp.float32),
    scratch_types=[
        pltpu.VMEM((2, EMB, VB), jnp.float32),
        pltpu.VMEM((VB * EMB,), jnp.float32),
        pltpu.SemaphoreType.DMA,
        pltpu.SemaphoreType.DMA,
    ],
)
def _sc_transpose(t3_hbm, tail_hbm, flat_hbm, in_v, out_v, sem_a, sem_b):
    wid = lax.axis_index("s") * 2 + lax.axis_index("c")
    lane = lax.iota(jnp.int32, 16)

    def fire(f, v0, buf, sem):
        for k in range(4):
            pltpu.async_copy(t3_hbm.at[f, pl.ds(k * 8, 8), pl.ds(v0, VB)],
                             in_v.at[buf, pl.ds(k * 8, 8), :], sem)

    def drain(f, v0, buf, sem):
        for k in range(4):
            pltpu.make_async_copy(t3_hbm.at[f, pl.ds(k * 8, 8), pl.ds(v0, VB)],
                                  in_v.at[buf, pl.ds(k * 8, 8), :], sem).wait()

    def transpose_store(f, v0, buf):
        def tr_body(vg, carry2):
            v = vg * 16
            for dv in range(16):
                col = jnp.full((16,), v + dv, jnp.int32)
                bufc = jnp.full((16,), buf, jnp.int32)
                lo = plsc.load_gather(in_v, [bufc, lane, col])
                hi = plsc.load_gather(in_v, [bufc, lane + 16, col])
                out_v[pl.ds((v + dv) * EMB, 16)] = lo
                out_v[pl.ds((v + dv) * EMB + 16, 16)] = hi
            return carry2

        lax.fori_loop(0, VB // 16, tr_body, 0)
        pltpu.sync_copy(out_v,
                        flat_hbm.at[pl.ds((f * VOCAB + v0) * EMB, VB * EMB)])

    def field_body(f, carry):
        @pl.when(wid < NFULL)
        def _():
            fire(f, wid * VB, 0, sem_a)

        def blk_body(k, carry2):
            vb = k * NW + wid
            nxt = vb + NW

            @pl.when(jnp.logical_and(nxt < NFULL, k % 2 == 0))
            def _():
                fire(f, nxt * VB, 1, sem_b)

            @pl.when(jnp.logical_and(nxt < NFULL, k % 2 == 1))
            def _():
                fire(f, nxt * VB, 0, sem_a)

            @pl.when(jnp.logical_and(vb < NFULL, k % 2 == 0))
            def _():
                drain(f, vb * VB, 0, sem_a)
                transpose_store(f, vb * VB, 0)

            @pl.when(jnp.logical_and(vb < NFULL, k % 2 == 1))
            def _():
                drain(f, vb * VB, 1, sem_b)
                transpose_store(f, vb * VB, 1)

            return carry2

        lax.fori_loop(0, KMAX, blk_body, 0)
        return carry

    lax.fori_loop(0, N_FIELDS, field_body, 0)

    @pl.when(wid < N_FIELDS)
    def _():
        pltpu.sync_copy(
            tail_hbm.at[pl.ds(wid * 32 * EMB, 32 * EMB)],
            flat_hbm.at[pl.ds((wid * VOCAB + VTAIL0) * EMB, 32 * EMB)])


@functools.partial(
    pl.kernel,
    mesh=_mesh,
    compiler_params=pltpu.CompilerParams(use_tc_tiling_on_sc=False),
    out_type=jax.ShapeDtypeStruct((J_ROWS, EMB), jnp.float32),
    scratch_types=[
        pltpu.VMEM((GCHUNK,), jnp.int32),
        pltpu.VMEM((GCHUNK,), jnp.int32),
        pltpu.VMEM((GCHUNK, EMB), jnp.float32),
        pltpu.SemaphoreType.DMA,
        pltpu.SemaphoreType.DMA,
    ],
)
def _sc_gather(idx_hbm, table_hbm, out_hbm, idx_v, jidx_v, rows_v, sem, sem2):
    wid = lax.axis_index("s") * 2 + lax.axis_index("c")
    base = wid * G_PER_W
    lane = lax.iota(jnp.int32, 16)

    def body(c, carry):
        off = base + c * GCHUNK
        pltpu.sync_copy(idx_hbm.at[pl.ds(off, GCHUNK)], idx_v)
        pltpu.async_copy(table_hbm.at[idx_v], rows_v, sem).wait()

        def jbody(g, carry2):
            r = off + g * 16 + lane
            b = lax.div(r, N_FIELDS)
            f = r - b * N_FIELDS
            j = ((lax.shift_right_logical(b, 3) * NCB
                  + lax.shift_right_logical(f, 2)) * 32
                 + (b & 7) * 4 + (f & 3))
            jidx_v[pl.ds(g * 16, 16)] = j
            return carry2

        lax.fori_loop(0, GCHUNK // 16, jbody, 0)
        pltpu.async_copy(rows_v, out_hbm.at[jidx_v], sem2).wait()
        return carry

    lax.fori_loop(0, NGCHUNK, body, 0)


BM = 2048


def _mlp_body(emb_ref, num_ref, w1e_ref, w1n_ref, b1_ref, w2_ref, b2_ref, out_ref):
    h = jnp.dot(num_ref[...], w1n_ref[...], preferred_element_type=jnp.float32)
    for cb in range(NCB):
        x = emb_ref[:, cb, :, :].reshape(BM, 128)
        if cb == NCB - 1:
            col = lax.broadcasted_iota(jnp.int32, (BM, 128), 1)
            x = jnp.where(col < 64, x, 0.0)
        h = h + jnp.dot(x, w1e_ref[cb], preferred_element_type=jnp.float32)
    h = jnp.maximum(h + b1_ref[...], 0.0)
    o = jnp.dot(h, w2_ref[...], preferred_element_type=jnp.float32) + b2_ref[...]
    out_ref[...] = jnp.maximum(o, 0.0)


_mlp = pl.pallas_call(
    _mlp_body,
    grid=(B // BM,),
    in_specs=[
        pl.BlockSpec((BM // 8, NCB, 8, 128), lambda i: (i, 0, 0, 0)),
        pl.BlockSpec((BM, NUM_PAD), lambda i: (i, 0)),
        pl.BlockSpec((NCB, 128, H1), lambda i: (0, 0, 0)),
        pl.BlockSpec((NUM_PAD, H1), lambda i: (0, 0)),
        pl.BlockSpec((1, H1), lambda i: (0, 0)),
        pl.BlockSpec((H1, H2), lambda i: (0, 0)),
        pl.BlockSpec((1, H2), lambda i: (0, 0)),
    ],
    out_specs=pl.BlockSpec((BM, H2), lambda i: (i, 0)),
    out_shape=jax.ShapeDtypeStruct((B, H2), jnp.float32),
)


def kernel(x_categorical, x_numerical, tables, bn_gamma, bn_beta, bn_mean, bn_var,
           W1, b1, W2, b2):
    t3 = jnp.transpose(tables, (0, 2, 1))
    tail = tables[:, VTAIL0:, :].reshape(-1)
    flat = _sc_transpose(t3, tail)
    flat2d = flat.reshape(N_FIELDS * VOCAB, EMB)

    x_cat = x_categorical.astype(jnp.int32)
    flat_idx = (x_cat + (jnp.arange(N_FIELDS, dtype=jnp.int32) * VOCAB)[None, :]
                ).reshape(-1)
    embt = _sc_gather(flat_idx, flat2d)
    emb4 = embt.reshape(B // 8, NCB, 8, 128)

    scale = bn_gamma * lax.rsqrt(bn_var + 1e-5)
    shift = bn_beta - bn_mean * scale
    W1e_T = jnp.zeros((NCB * 128, H1), jnp.float32).at[:832].set(W1[:, :832].T)
    W1e_T = W1e_T.reshape(NCB, 128, H1)
    W1n = W1[:, 832:]
    W1n_T = (W1n * scale[None, :]).T
    W1n_T_pad = jnp.zeros((NUM_PAD, H1), jnp.float32).at[:NUM].set(W1n_T)
    b1_eff = (b1 + W1n @ shift).reshape(1, H1)
    x_num_pad = jnp.zeros((B, NUM_PAD), jnp.float32).at[:, :NUM].set(x_numerical)

    return _mlp(emb4, x_num_pad, W1e_T, W1n_T_pad, b1_eff, W2.T, b2.reshape(1, H2))


# speedup vs baseline: 1.0964x; 1.0964x over previous
import functools

import jax
import jax.numpy as jnp
from jax import lax
from jax.experimental import pallas as pl
from jax.experimental.pallas import tpu as pltpu
from jax.experimental.pallas import tpu_sc as plsc

B = 16384
N_FIELDS = 26
VOCAB = 100000
EMB = 32
NUM = 13
H1 = 256
H2 = 128
NUM_PAD = 64

N_ROWS = B * N_FIELDS
NW = 32

VB = 128
NFULL = VOCAB // VB
VTAIL0 = NFULL * VB
NBLK = N_FIELDS * NFULL
KMAX = (NBLK + NW - 1) // NW
VSTRIDE = VOCAB
FLAT_LEN = N_FIELDS * VSTRIDE * EMB

G_PER_W = N_ROWS // NW
GCHUNK = 1664
NGCHUNK = G_PER_W // GCHUNK

NCB = 7
J_ROWS = (B // 8) * NCB * 32

_mesh = plsc.VectorSubcoreMesh(core_axis_name="c", subcore_axis_name="s")


_BLK_W = VB * EMB


@functools.partial(
    pl.kernel,
    mesh=_mesh,
    compiler_params=pltpu.CompilerParams(needs_layout_passes=False),
    out_type=jax.ShapeDtypeStruct((FLAT_LEN,), jnp.float32),
    scratch_types=[
        pltpu.VMEM((4, EMB, VB), jnp.float32),
        pltpu.VMEM((2 * _BLK_W,), jnp.float32),
        pltpu.SemaphoreType.DMA,
        pltpu.SemaphoreType.DMA,
        pltpu.SemaphoreType.DMA,
        pltpu.SemaphoreType.DMA,
        pltpu.SemaphoreType.DMA,
        pltpu.SemaphoreType.DMA,
    ],
)
def _sc_transpose(t3_hbm, tail_hbm, flat_hbm, in_v, out_v,
                  s0, s1, s2, s3, so0, so1):
    wid = lax.axis_index("s") * 2 + lax.axis_index("c")
    lane = lax.iota(jnp.int32, 16)
    in_sems = (s0, s1, s2, s3)

    def fire(u, buf, sem):
        f = lax.div(u, NFULL)
        v0 = lax.rem(u, NFULL) * VB
        for k in range(4):
            pltpu.async_copy(t3_hbm.at[f, pl.ds(k * 8, 8), pl.ds(v0, VB)],
                             in_v.at[buf, pl.ds(k * 8, 8), :], sem)

    def drain(u, buf, sem):
        f = lax.div(u, NFULL)
        v0 = lax.rem(u, NFULL) * VB
        for k in range(4):
            pltpu.make_async_copy(t3_hbm.at[f, pl.ds(k * 8, 8), pl.ds(v0, VB)],
                                  in_v.at[buf, pl.ds(k * 8, 8), :], sem).wait()

    def wait_out(sem):
        pltpu.make_async_copy(out_v.at[pl.ds(0, _BLK_W)],
                              flat_hbm.at[pl.ds(0, _BLK_W)], sem).wait()

    for j in range(3):
        fire(j * NW + wid, j, in_sems[j])

    def body(k, carry):
        km4 = lax.rem(k, 4)
        km2 = lax.rem(k, 2)

        @pl.when(jnp.logical_and(k >= 2, km2 == 0))
        def _():
            wait_out(so0)

        @pl.when(jnp.logical_and(k >= 2, km2 == 1))
        def _():
            wait_out(so1)

        un = (k + 3) * NW + wid
        for p in range(4):
            @pl.when(jnp.logical_and(un < NBLK, km4 == (p + 1) % 4))
            def _(p=p):
                fire(un, p, in_sems[p])

        u = k * NW + wid

        @pl.when(u < NBLK)
        def _():
            for p in range(4):
                @pl.when(km4 == p)
                def _(p=p):
                    drain(u, p, in_sems[p])

            obase = km2 * _BLK_W
            bufc = jnp.full((16,), km4, jnp.int32)

            def tr_body(vg, carry2):
                v = vg * 16
                for dv in range(16):
                    col = jnp.full((16,), v + dv, jnp.int32)
                    lo = plsc.load_gather(in_v, [bufc, lane, col])
                    hi = plsc.load_gather(in_v, [bufc, lane + 16, col])
                    out_v[pl.ds(obase + (v + dv) * EMB, 16)] = lo
                    out_v[pl.ds(obase + (v + dv) * EMB + 16, 16)] = hi
                return carry2

            lax.fori_loop(0, VB // 16, tr_body, 0)

            f = lax.div(u, NFULL)
            v0 = lax.rem(u, NFULL) * VB
            dst = (f * VOCAB + v0) * EMB

            @pl.when(km2 == 0)
            def _():
                pltpu.async_copy(out_v.at[pl.ds(obase, _BLK_W)],
                                 flat_hbm.at[pl.ds(dst, _BLK_W)], so0)

            @pl.when(km2 == 1)
            def _():
                pltpu.async_copy(out_v.at[pl.ds(obase, _BLK_W)],
                                 flat_hbm.at[pl.ds(dst, _BLK_W)], so1)

        return carry

    lax.fori_loop(0, KMAX, body, 0)

    wait_out(so1)

    @pl.when((KMAX - 1) * NW + wid < NBLK)
    def _():
        wait_out(so0)

    @pl.when(wid < N_FIELDS)
    def _():
        pltpu.sync_copy(
            tail_hbm.at[pl.ds(wid * 32 * EMB, 32 * EMB)],
            flat_hbm.at[pl.ds((wid * VOCAB + VTAIL0) * EMB, 32 * EMB)])


@functools.partial(
    pl.kernel,
    mesh=_mesh,
    compiler_params=pltpu.CompilerParams(use_tc_tiling_on_sc=False),
    out_type=jax.ShapeDtypeStruct((J_ROWS, EMB), jnp.float32),
    scratch_types=[
        pltpu.VMEM((GCHUNK,), jnp.int32),
        pltpu.VMEM((GCHUNK,), jnp.int32),
        pltpu.VMEM((GCHUNK, EMB), jnp.float32),
        pltpu.SemaphoreType.DMA,
        pltpu.SemaphoreType.DMA,
    ],
)
def _sc_gather(idx_hbm, table_hbm, out_hbm, idx_v, jidx_v, rows_v, sem, sem2):
    wid = lax.axis_index("s") * 2 + lax.axis_index("c")
    base = wid * G_PER_W
    lane = lax.iota(jnp.int32, 16)

    def body(c, carry):
        off = base + c * GCHUNK
        pltpu.sync_copy(idx_hbm.at[pl.ds(off, GCHUNK)], idx_v)
        pltpu.async_copy(table_hbm.at[idx_v], rows_v, sem).wait()

        def jbody(g, carry2):
            r = off + g * 16 + lane
            b = lax.div(r, N_FIELDS)
            f = r - b * N_FIELDS
            j = ((lax.shift_right_logical(b, 3) * NCB
                  + lax.shift_right_logical(f, 2)) * 32
                 + (b & 7) * 4 + (f & 3))
            jidx_v[pl.ds(g * 16, 16)] = j
            return carry2

        lax.fori_loop(0, GCHUNK // 16, jbody, 0)
        pltpu.async_copy(rows_v, out_hbm.at[jidx_v], sem2).wait()
        return carry

    lax.fori_loop(0, NGCHUNK, body, 0)


BM = 2048


def _mlp_body(emb_ref, num_ref, w1e_ref, w1n_ref, b1_ref, w2_ref, b2_ref, out_ref):
    h = jnp.dot(num_ref[...], w1n_ref[...], preferred_element_type=jnp.float32)
    for cb in range(NCB):
        x = emb_ref[:, cb, :, :].reshape(BM, 128)
        if cb == NCB - 1:
            col = lax.broadcasted_iota(jnp.int32, (BM, 128), 1)
            x = jnp.where(col < 64, x, 0.0)
        h = h + jnp.dot(x, w1e_ref[cb], preferred_element_type=jnp.float32)
    h = jnp.maximum(h + b1_ref[...], 0.0)
    o = jnp.dot(h, w2_ref[...], preferred_element_type=jnp.float32) + b2_ref[...]
    out_ref[...] = jnp.maximum(o, 0.0)


_mlp = pl.pallas_call(
    _mlp_body,
    grid=(B // BM,),
    in_specs=[
        pl.BlockSpec((BM // 8, NCB, 8, 128), lambda i: (i, 0, 0, 0)),
        pl.BlockSpec((BM, NUM_PAD), lambda i: (i, 0)),
        pl.BlockSpec((NCB, 128, H1), lambda i: (0, 0, 0)),
        pl.BlockSpec((NUM_PAD, H1), lambda i: (0, 0)),
        pl.BlockSpec((1, H1), lambda i: (0, 0)),
        pl.BlockSpec((H1, H2), lambda i: (0, 0)),
        pl.BlockSpec((1, H2), lambda i: (0, 0)),
    ],
    out_specs=pl.BlockSpec((BM, H2), lambda i: (i, 0)),
    out_shape=jax.ShapeDtypeStruct((B, H2), jnp.float32),
)


def kernel(x_categorical, x_numerical, tables, bn_gamma, bn_beta, bn_mean, bn_var,
           W1, b1, W2, b2):
    t3 = jnp.transpose(tables, (0, 2, 1))
    tail = tables[:, VTAIL0:, :].reshape(-1)
    flat = _sc_transpose(t3, tail)
    flat2d = flat.reshape(N_FIELDS * VSTRIDE, EMB)

    x_cat = x_categorical.astype(jnp.int32)
    flat_idx = (x_cat + (jnp.arange(N_FIELDS, dtype=jnp.int32) * VSTRIDE)[None, :]
                ).reshape(-1)
    embt = _sc_gather(flat_idx, flat2d)
    emb4 = embt.reshape(B // 8, NCB, 8, 128)

    scale = bn_gamma * lax.rsqrt(bn_var + 1e-5)
    shift = bn_beta - bn_mean * scale
    W1e_T = jnp.zeros((NCB * 128, H1), jnp.float32).at[:832].set(W1[:, :832].T)
    W1e_T = W1e_T.reshape(NCB, 128, H1)
    W1n = W1[:, 832:]
    W1n_T = (W1n * scale[None, :]).T
    W1n_T_pad = jnp.zeros((NUM_PAD, H1), jnp.float32).at[:NUM].set(W1n_T)
    b1_eff = (b1 + W1n @ shift).reshape(1, H1)
    x_num_pad = jnp.zeros((B, NUM_PAD), jnp.float32).at[:, :NUM].set(x_numerical)

    return _mlp(emb4, x_num_pad, W1e_T, W1n_T_pad, b1_eff, W2.T, b2.reshape(1, H2))


# speedup vs baseline: 1.6195x; 1.4770x over previous
import functools

import jax
import jax.numpy as jnp
from jax import lax
from jax.experimental import pallas as pl
from jax.experimental.pallas import tpu as pltpu
from jax.experimental.pallas import tpu_sc as plsc

B = 16384
N_FIELDS = 26
VOCAB = 100000
EMB = 32
NUM = 13
H1 = 256
H2 = 128
NUM_PAD = 64

N_ROWS = B * N_FIELDS
NW = 32

VB = 128
NFULL = VOCAB // VB
VTAIL0 = NFULL * VB
NBLK = N_FIELDS * NFULL
KMAX = (NBLK + NW - 1) // NW
VSTRIDE = VOCAB
FLAT_LEN = N_FIELDS * VSTRIDE * EMB

G_PER_W = N_ROWS // NW
GCHUNK = 1664
NGCHUNK = G_PER_W // GCHUNK

NCB = 7
J_ROWS = (B // 8) * NCB * 32

_mesh = plsc.VectorSubcoreMesh(core_axis_name="c", subcore_axis_name="s")


_BLK_W = VB * EMB


@functools.partial(
    pl.kernel,
    mesh=_mesh,
    compiler_params=pltpu.CompilerParams(needs_layout_passes=False),
    out_type=jax.ShapeDtypeStruct((FLAT_LEN,), jnp.float32),
    scratch_types=[
        pltpu.VMEM((4, EMB, VB), jnp.float32),
        pltpu.VMEM((2 * _BLK_W,), jnp.float32),
        pltpu.SemaphoreType.DMA,
        pltpu.SemaphoreType.DMA,
        pltpu.SemaphoreType.DMA,
        pltpu.SemaphoreType.DMA,
        pltpu.SemaphoreType.DMA,
        pltpu.SemaphoreType.DMA,
    ],
)
def _sc_transpose(t3_hbm, tail_hbm, flat_hbm, in_v, out_v,
                  s0, s1, s2, s3, so0, so1):
    wid = lax.axis_index("s") * 2 + lax.axis_index("c")
    lane = lax.iota(jnp.int32, 16)
    in_sems = (s0, s1, s2, s3)

    def fire(u, buf, sem):
        f = lax.div(u, NFULL)
        v0 = lax.rem(u, NFULL) * VB
        for k in range(4):
            pltpu.async_copy(t3_hbm.at[f, pl.ds(k * 8, 8), pl.ds(v0, VB)],
                             in_v.at[buf, pl.ds(k * 8, 8), :], sem)

    def drain(u, buf, sem):
        f = lax.div(u, NFULL)
        v0 = lax.rem(u, NFULL) * VB
        for k in range(4):
            pltpu.make_async_copy(t3_hbm.at[f, pl.ds(k * 8, 8), pl.ds(v0, VB)],
                                  in_v.at[buf, pl.ds(k * 8, 8), :], sem).wait()

    def wait_out(sem):
        pltpu.make_async_copy(out_v.at[pl.ds(0, _BLK_W)],
                              flat_hbm.at[pl.ds(0, _BLK_W)], sem).wait()

    for j in range(3):
        fire(j * NW + wid, j, in_sems[j])

    def body(k, carry):
        km4 = lax.rem(k, 4)
        km2 = lax.rem(k, 2)

        @pl.when(jnp.logical_and(k >= 2, km2 == 0))
        def _():
            wait_out(so0)

        @pl.when(jnp.logical_and(k >= 2, km2 == 1))
        def _():
            wait_out(so1)

        un = (k + 3) * NW + wid
        for p in range(4):
            @pl.when(jnp.logical_and(un < NBLK, km4 == (p + 1) % 4))
            def _(p=p):
                fire(un, p, in_sems[p])

        u = k * NW + wid

        @pl.when(u < NBLK)
        def _():
            for p in range(4):
                @pl.when(km4 == p)
                def _(p=p):
                    drain(u, p, in_sems[p])

            obase = km2 * _BLK_W
            bufc = jnp.full((16,), km4, jnp.int32)

            def tr_body(vg, carry2):
                v = vg * 16
                for dv in range(16):
                    col = jnp.full((16,), v + dv, jnp.int32)
                    lo = plsc.load_gather(in_v, [bufc, lane, col])
                    hi = plsc.load_gather(in_v, [bufc, lane + 16, col])
                    out_v[pl.ds(obase + (v + dv) * EMB, 16)] = lo
                    out_v[pl.ds(obase + (v + dv) * EMB + 16, 16)] = hi
                return carry2

            lax.fori_loop(0, VB // 16, tr_body, 0)

            f = lax.div(u, NFULL)
            v0 = lax.rem(u, NFULL) * VB
            dst = (f * VOCAB + v0) * EMB

            @pl.when(km2 == 0)
            def _():
                pltpu.async_copy(out_v.at[pl.ds(obase, _BLK_W)],
                                 flat_hbm.at[pl.ds(dst, _BLK_W)], so0)

            @pl.when(km2 == 1)
            def _():
                pltpu.async_copy(out_v.at[pl.ds(obase, _BLK_W)],
                                 flat_hbm.at[pl.ds(dst, _BLK_W)], so1)

        return carry

    lax.fori_loop(0, KMAX, body, 0)

    wait_out(so1)

    @pl.when((KMAX - 1) * NW + wid < NBLK)
    def _():
        wait_out(so0)

    @pl.when(wid < N_FIELDS)
    def _():
        pltpu.sync_copy(
            tail_hbm.at[pl.ds(wid * 32 * EMB, 32 * EMB)],
            flat_hbm.at[pl.ds((wid * VOCAB + VTAIL0) * EMB, 32 * EMB)])


@functools.partial(
    pl.kernel,
    mesh=_mesh,
    compiler_params=pltpu.CompilerParams(use_tc_tiling_on_sc=False),
    out_type=jax.ShapeDtypeStruct((J_ROWS, EMB), jnp.float32),
    scratch_types=[
        pltpu.VMEM((GCHUNK,), jnp.int32),
        pltpu.VMEM((GCHUNK,), jnp.int32),
        pltpu.VMEM((GCHUNK, EMB), jnp.float32),
        pltpu.SemaphoreType.DMA,
        pltpu.SemaphoreType.DMA,
    ],
)
def _sc_gather(idx_hbm, table_hbm, out_hbm, idx_v, jidx_v, rows_v, sem, sem2):
    wid = lax.axis_index("s") * 2 + lax.axis_index("c")
    base = wid * G_PER_W
    lane = lax.iota(jnp.int32, 16)

    def body(c, carry):
        off = base + c * GCHUNK
        pltpu.sync_copy(idx_hbm.at[pl.ds(off, GCHUNK)], idx_v)
        pltpu.async_copy(table_hbm.at[idx_v], rows_v, sem).wait()

        def jbody(g, carry2):
            r = off + g * 16 + lane
            b = lax.div(r, N_FIELDS)
            f = r - b * N_FIELDS
            j = ((lax.shift_right_logical(b, 3) * NCB
                  + lax.shift_right_logical(f, 2)) * 32
                 + (b & 7) * 4 + (f & 3))
            jidx_v[pl.ds(g * 16, 16)] = j
            return carry2

        lax.fori_loop(0, GCHUNK // 16, jbody, 0)
        pltpu.async_copy(rows_v, out_hbm.at[jidx_v], sem2).wait()
        return carry

    lax.fori_loop(0, NGCHUNK, body, 0)


BM = 2048


def _mlp_body(emb_ref, num_ref, w1e_ref, w1n_ref, b1_ref, w2_ref, b2_ref, out_ref):
    h = jnp.dot(num_ref[...], w1n_ref[...], preferred_element_type=jnp.float32)
    for cb in range(NCB):
        x = emb_ref[:, cb, :, :].reshape(BM, 128)
        if cb == NCB - 1:
            col = lax.broadcasted_iota(jnp.int32, (BM, 128), 1)
            x = jnp.where(col < 64, x, 0.0)
        h = h + jnp.dot(x, w1e_ref[cb], preferred_element_type=jnp.float32)
    h = jnp.maximum(h + b1_ref[...], 0.0)
    o = jnp.dot(h, w2_ref[...], preferred_element_type=jnp.float32) + b2_ref[...]
    out_ref[...] = jnp.maximum(o, 0.0)


_mlp = pl.pallas_call(
    _mlp_body,
    grid=(B // BM,),
    in_specs=[
        pl.BlockSpec((BM // 8, NCB, 8, 128), lambda i: (i, 0, 0, 0)),
        pl.BlockSpec((BM, NUM_PAD), lambda i: (i, 0)),
        pl.BlockSpec((NCB, 128, H1), lambda i: (0, 0, 0)),
        pl.BlockSpec((NUM_PAD, H1), lambda i: (0, 0)),
        pl.BlockSpec((1, H1), lambda i: (0, 0)),
        pl.BlockSpec((H1, H2), lambda i: (0, 0)),
        pl.BlockSpec((1, H2), lambda i: (0, 0)),
    ],
    out_specs=pl.BlockSpec((BM, H2), lambda i: (i, 0)),
    out_shape=jax.ShapeDtypeStruct((B, H2), jnp.float32),
)


def kernel(x_categorical, x_numerical, tables, bn_gamma, bn_beta, bn_mean, bn_var,
           W1, b1, W2, b2):
    flat2d = tables.reshape(N_FIELDS * VSTRIDE, EMB)

    x_cat = x_categorical.astype(jnp.int32)
    flat_idx = (x_cat + (jnp.arange(N_FIELDS, dtype=jnp.int32) * VSTRIDE)[None, :]
                ).reshape(-1)
    embt = _sc_gather(flat_idx, flat2d)
    emb4 = embt.reshape(B // 8, NCB, 8, 128)

    scale = bn_gamma * lax.rsqrt(bn_var + 1e-5)
    shift = bn_beta - bn_mean * scale
    W1e_T = jnp.zeros((NCB * 128, H1), jnp.float32).at[:832].set(W1[:, :832].T)
    W1e_T = W1e_T.reshape(NCB, 128, H1)
    W1n = W1[:, 832:]
    W1n_T = (W1n * scale[None, :]).T
    W1n_T_pad = jnp.zeros((NUM_PAD, H1), jnp.float32).at[:NUM].set(W1n_T)
    b1_eff = (b1 + W1n @ shift).reshape(1, H1)
    x_num_pad = jnp.zeros((B, NUM_PAD), jnp.float32).at[:, :NUM].set(x_numerical)

    return _mlp(emb4, x_num_pad, W1e_T, W1n_T_pad, b1_eff, W2.T, b2.reshape(1, H2))
